# Initial kernel scaffold; baseline (speedup 1.0000x reference)
import jax
import jax.numpy as jnp
from jax.experimental import pallas as pl


def kernel(key, value, key_cache, value_cache):
    raise NotImplementedError("write your pallas kernel here")


# speedup vs baseline: 48.6729x; 48.6729x over previous
import jax
import jax.numpy as jnp
from jax.experimental import pallas as pl


def _copy_kernel(k_ref, v_ref, ko_ref, vo_ref):
    ko_ref[...] = k_ref[...]
    vo_ref[...] = v_ref[...]


def kernel(key, value, key_cache, value_cache):
    del key_cache, value_cache
    out_shape = jax.ShapeDtypeStruct(key.shape, key.dtype)
    return pl.pallas_call(
        _copy_kernel,
        out_shape=(out_shape, out_shape),
    )(key, value)
